# baseline (speedup 1.0000x reference)
import functools

import jax
import jax.numpy as jnp
from jax import lax
from jax.experimental import pallas as pl
from jax.experimental.pallas import tpu as pltpu

_BINS = 30
_SCALE = 30 - 0.0001
_LANES = 128
_BR = 1024
_CH = 8
_LEVELS = 12


def _bins_of(x, t):
    g = jnp.abs(jax.nn.sigmoid(x) - t)
    return jnp.floor(g * _SCALE).astype(jnp.int32)


def _csa(a, b, c):
    u = a ^ b
    return u ^ c, (a & b) | (u & c)


_QS = 1024.0


def _hist_kernel(x_ref, t_ref, beta_ref, bidx_ref, lq_ref, planes_ref, *, nblocks, tot, ncols):
    i = pl.program_id(0)

    @pl.when(i == 0)
    def _init():
        planes_ref[...] = jnp.zeros_like(planes_ref)

    x = x_ref[...]
    t = t_ref[...]
    bb = _bins_of(x, t)
    bidx_ref[...] = bb.astype(jnp.int8)
    lval = jnp.maximum(x, 0.0) - x * t + jnp.log1p(jnp.exp(-jnp.abs(x)))
    lq_ref[...] = jnp.floor(lval * _QS + 0.5).astype(jnp.int16)
    v = jnp.left_shift(jnp.int32(1), bb)

    vals = {0: [v[k * _CH:(k + 1) * _CH, :] for k in range(_BR // _CH)]}
    j = 0
    while j in vals:
        lv = vals[j]
        carries = []
        while len(lv) >= 3:
            s, co = _csa(lv.pop(), lv.pop(), lv.pop())
            lv.append(s)
            carries.append(co)
        if len(lv) == 2:
            a0, a1 = lv
            lv = [a0 ^ a1]
            carries.append(a0 & a1)
        if carries:
            vals[j + 1] = carries
        if lv:
            carry = lv[0]
            for lvl in range(j, _LEVELS):
                old = planes_ref[lvl]
                planes_ref[lvl] = old ^ carry
                carry = old & carry
        j += 1

    @pl.when(i == nblocks - 1)
    def _extract():
        li = lax.broadcasted_iota(jnp.int32, (1, _LANES), 1)
        vec = jnp.zeros((1, _LANES), jnp.float32)
        for k in range(_BINS):
            c = jnp.float32(0.0)
            for lvl in range(_LEVELS):
                bits = (planes_ref[lvl] >> k) & 1
                c = c + jnp.float32(1 << lvl) * jnp.sum(bits).astype(jnp.float32)
            vec = vec + jnp.where(li == k, c, 0.0)
        ne = jnp.sum(jnp.where((li < _BINS) & (vec > 0), 1.0, 0.0))
        beta = tot / jnp.clip(vec * ne, 0.0001, None)
        beta_ref[...] = beta * (1.0 / (_QS * ncols))


def _loss_kernel(beta_ref, bidx_ref, lq_ref, out_ref):
    bb = bidx_ref[...].astype(jnp.int32)
    lq = lq_ref[...].astype(jnp.float32)
    tab = jnp.broadcast_to(beta_ref[...][:, :32], (bb.shape[0], 32))
    w = jnp.take_along_axis(tab, bb, axis=1)
    out_ref[...] = jnp.sum(w * lq, axis=1)


def kernel(logits, target):
    rows, cols = logits.shape
    nblocks = rows // _BR
    tot = float(logits.size)

    beta, bidx, lq = pl.pallas_call(
        functools.partial(_hist_kernel, nblocks=nblocks, tot=tot, ncols=float(cols)),
        grid=(nblocks,),
        in_specs=[
            pl.BlockSpec((_BR, cols), lambda i: (i, 0)),
            pl.BlockSpec((_BR, cols), lambda i: (i, 0)),
        ],
        out_specs=[
            pl.BlockSpec((1, _LANES), lambda i: (0, 0)),
            pl.BlockSpec((_BR, cols), lambda i: (i, 0)),
            pl.BlockSpec((_BR, cols), lambda i: (i, 0)),
        ],
        out_shape=[
            jax.ShapeDtypeStruct((1, _LANES), jnp.float32),
            jax.ShapeDtypeStruct((rows, cols), jnp.int8),
            jax.ShapeDtypeStruct((rows, cols), jnp.int16),
        ],
        scratch_shapes=[pltpu.VMEM((_LEVELS, _CH, cols), jnp.int32)],
        compiler_params=pltpu.CompilerParams(
            dimension_semantics=("arbitrary",),
        ),
    )(logits, target)

    out = pl.pallas_call(
        _loss_kernel,
        grid=(nblocks,),
        in_specs=[
            pl.BlockSpec((1, _LANES), lambda i: (0, 0)),
            pl.BlockSpec((_BR, cols), lambda i: (i, 0)),
            pl.BlockSpec((_BR, cols), lambda i: (i, 0)),
        ],
        out_specs=pl.BlockSpec((_BR,), lambda i: (i,)),
        out_shape=jax.ShapeDtypeStruct((rows,), jnp.float32),
        compiler_params=pltpu.CompilerParams(
            dimension_semantics=("arbitrary",),
        ),
    )(beta, bidx, lq)
    return out


# speedup vs baseline: 1.2397x; 1.2397x over previous
import functools

import jax
import jax.numpy as jnp
from jax import lax
from jax.experimental import pallas as pl
from jax.experimental.pallas import tpu as pltpu

_BINS = 30
_SCALE = 30 - 0.0001
_LANES = 128
_BR = 2048
_CH = 8
_LEVELS = 12


def _bins_of(x, t):
    g = jnp.abs(jax.nn.sigmoid(x) - t)
    return jnp.floor(g * _SCALE).astype(jnp.int32)


def _csa(a, b, c):
    u = a ^ b
    return u ^ c, (a & b) | (u & c)


def _hist_kernel(x_ref, t_ref, beta_ref, bidx_ref, planes_ref, *, nblocks, tot):
    i = pl.program_id(0)

    @pl.when(i == 0)
    def _init():
        planes_ref[...] = jnp.zeros_like(planes_ref)

    bb = _bins_of(x_ref[...], t_ref[...])
    bidx_ref[...] = bb.astype(jnp.int8)
    v = jnp.left_shift(jnp.int32(1), bb)

    vals = {0: [v[k * _CH:(k + 1) * _CH, :] for k in range(_BR // _CH)]}
    j = 0
    while j in vals:
        lv = vals[j]
        carries = []
        while len(lv) >= 3:
            s, co = _csa(lv.pop(), lv.pop(), lv.pop())
            lv.append(s)
            carries.append(co)
        if len(lv) == 2:
            a0, a1 = lv
            lv = [a0 ^ a1]
            carries.append(a0 & a1)
        if carries:
            vals[j + 1] = carries
        if lv:
            carry = lv[0]
            for lvl in range(j, _LEVELS):
                old = planes_ref[lvl]
                planes_ref[lvl] = old ^ carry
                carry = old & carry
        j += 1

    @pl.when(i == nblocks - 1)
    def _extract():
        li = lax.broadcasted_iota(jnp.int32, (1, _LANES), 1)
        vec = jnp.zeros((1, _LANES), jnp.float32)
        for k in range(_BINS):
            c = jnp.float32(0.0)
            for lvl in range(_LEVELS):
                bits = (planes_ref[lvl] >> k) & 1
                c = c + jnp.float32(1 << lvl) * jnp.sum(bits).astype(jnp.float32)
            vec = vec + jnp.where(li == k, c, 0.0)
        ne = jnp.sum(jnp.where((li < _BINS) & (vec > 0), 1.0, 0.0))
        beta_ref[...] = tot / jnp.clip(vec * ne, 0.0001, None)


def _loss_kernel(beta_ref, bidx_ref, x_ref, t_ref, out_ref):
    x = x_ref[...]
    t = t_ref[...]
    bb = bidx_ref[...].astype(jnp.int32)
    tab = jnp.broadcast_to(beta_ref[...][:, :32], (x.shape[0], 32))
    w = jnp.take_along_axis(tab, bb, axis=1)
    loss = w * (jnp.maximum(x, 0.0) - x * t + jnp.log1p(jnp.exp(-jnp.abs(x))))
    out_ref[...] = jnp.mean(loss, axis=1)


def kernel(logits, target):
    rows, cols = logits.shape
    nblocks = rows // _BR
    tot = float(logits.size)

    beta, bidx = pl.pallas_call(
        functools.partial(_hist_kernel, nblocks=nblocks, tot=tot),
        grid=(nblocks,),
        in_specs=[
            pl.BlockSpec((_BR, cols), lambda i: (i, 0)),
            pl.BlockSpec((_BR, cols), lambda i: (i, 0)),
        ],
        out_specs=[
            pl.BlockSpec((1, _LANES), lambda i: (0, 0)),
            pl.BlockSpec((_BR, cols), lambda i: (i, 0)),
        ],
        out_shape=[
            jax.ShapeDtypeStruct((1, _LANES), jnp.float32),
            jax.ShapeDtypeStruct((rows, cols), jnp.int8),
        ],
        scratch_shapes=[pltpu.VMEM((_LEVELS, _CH, cols), jnp.int32)],
        compiler_params=pltpu.CompilerParams(
            dimension_semantics=("arbitrary",),
        ),
    )(logits, target)

    out = pl.pallas_call(
        _loss_kernel,
        grid=(nblocks,),
        in_specs=[
            pl.BlockSpec((1, _LANES), lambda i: (0, 0)),
            pl.BlockSpec((_BR, cols), lambda i: (i, 0)),
            pl.BlockSpec((_BR, cols), lambda i: (i, 0)),
            pl.BlockSpec((_BR, cols), lambda i: (i, 0)),
        ],
        out_specs=pl.BlockSpec((_BR,), lambda i: (i,)),
        out_shape=jax.ShapeDtypeStruct((rows,), jnp.float32),
        compiler_params=pltpu.CompilerParams(
            dimension_semantics=("arbitrary",),
        ),
    )(beta, bidx, logits, target)
    return out


# speedup vs baseline: 1.3341x; 1.0762x over previous
import functools

import jax
import jax.numpy as jnp
from jax import lax
from jax.experimental import pallas as pl
from jax.experimental.pallas import tpu as pltpu

_BINS = 30
_SCALE = 30 - 0.0001
_LANES = 128
_BR = 1024
_CH = 8
_LEVELS = 12


def _bins_of(x, t):
    g = jnp.abs(jax.nn.sigmoid(x) - t)
    return jnp.floor(g * _SCALE).astype(jnp.int32)


def _csa(a, b, c):
    u = a ^ b
    return u ^ c, (a & b) | (u & c)


def _hist_kernel(x_ref, t_ref, beta_ref, bidx_ref, planes_ref, *, nblocks, tot):
    i = pl.program_id(0)

    @pl.when(i == 0)
    def _init():
        planes_ref[...] = jnp.zeros_like(planes_ref)

    bb = _bins_of(x_ref[...], t_ref[...])
    bidx_ref[...] = bb.astype(jnp.int8)
    v = jnp.left_shift(jnp.int32(1), bb)

    vals = {0: [v[k * _CH:(k + 1) * _CH, :] for k in range(_BR // _CH)]}
    j = 0
    while j in vals:
        lv = vals[j]
        carries = []
        while len(lv) >= 3:
            s, co = _csa(lv.pop(), lv.pop(), lv.pop())
            lv.append(s)
            carries.append(co)
        if len(lv) == 2:
            a0, a1 = lv
            lv = [a0 ^ a1]
            carries.append(a0 & a1)
        if carries:
            vals[j + 1] = carries
        if lv:
            carry = lv[0]
            for lvl in range(j, _LEVELS):
                old = planes_ref[lvl]
                planes_ref[lvl] = old ^ carry
                carry = old & carry
        j += 1

    @pl.when(i == nblocks - 1)
    def _extract():
        li = lax.broadcasted_iota(jnp.int32, (1, _LANES), 1)
        vec = jnp.zeros((1, _LANES), jnp.float32)
        for k in range(_BINS):
            c = jnp.float32(0.0)
            for lvl in range(_LEVELS):
                bits = (planes_ref[lvl] >> k) & 1
                c = c + jnp.float32(1 << lvl) * jnp.sum(bits).astype(jnp.float32)
            vec = vec + jnp.where(li == k, c, 0.0)
        ne = jnp.sum(jnp.where((li < _BINS) & (vec > 0), 1.0, 0.0))
        beta_ref[...] = tot / jnp.clip(vec * ne, 0.0001, None)


def _loss_kernel(beta_ref, bidx_ref, x_ref, t_ref, out_ref):
    x = x_ref[...]
    t = t_ref[...]
    bb = bidx_ref[...].astype(jnp.int32)
    tab = jnp.broadcast_to(beta_ref[...][:, :32], (x.shape[0], 32))
    w = jnp.take_along_axis(tab, bb, axis=1)
    loss = w * (jnp.maximum(x, 0.0) - x * t + jnp.log1p(jnp.exp(-jnp.abs(x))))
    ones = jnp.full((x.shape[1], 1), 1.0 / x.shape[1], dtype=jnp.float32)
    out_ref[...] = lax.dot_general(
        loss, ones, (((1,), (0,)), ((), ())),
        preferred_element_type=jnp.float32,
    )[:, 0]


def kernel(logits, target):
    rows, cols = logits.shape
    nblocks = rows // _BR
    tot = float(logits.size)

    beta, bidx = pl.pallas_call(
        functools.partial(_hist_kernel, nblocks=nblocks, tot=tot),
        grid=(nblocks,),
        in_specs=[
            pl.BlockSpec((_BR, cols), lambda i: (i, 0)),
            pl.BlockSpec((_BR, cols), lambda i: (i, 0)),
        ],
        out_specs=[
            pl.BlockSpec((1, _LANES), lambda i: (0, 0)),
            pl.BlockSpec((_BR, cols), lambda i: (i, 0)),
        ],
        out_shape=[
            jax.ShapeDtypeStruct((1, _LANES), jnp.float32),
            jax.ShapeDtypeStruct((rows, cols), jnp.int8),
        ],
        scratch_shapes=[pltpu.VMEM((_LEVELS, _CH, cols), jnp.int32)],
        compiler_params=pltpu.CompilerParams(
            dimension_semantics=("arbitrary",),
        ),
    )(logits, target)

    out = pl.pallas_call(
        _loss_kernel,
        grid=(nblocks,),
        in_specs=[
            pl.BlockSpec((1, _LANES), lambda i: (0, 0)),
            pl.BlockSpec((_BR, cols), lambda i: (i, 0)),
            pl.BlockSpec((_BR, cols), lambda i: (i, 0)),
            pl.BlockSpec((_BR, cols), lambda i: (i, 0)),
        ],
        out_specs=pl.BlockSpec((_BR,), lambda i: (i,)),
        out_shape=jax.ShapeDtypeStruct((rows,), jnp.float32),
        compiler_params=pltpu.CompilerParams(
            dimension_semantics=("arbitrary",),
        ),
    )(beta, bidx, logits, target)
    return out
